# Initial kernel scaffold; baseline (speedup 1.0000x reference)
import jax
import jax.numpy as jnp
from jax.experimental import pallas as pl


def kernel(X, edge_index, W, a_self, a_neigh, bias):
    raise NotImplementedError("write your pallas kernel here")


# speedup vs baseline: 49.2670x; 49.2670x over previous
import jax
import jax.numpy as jnp
from jax import lax
from jax.experimental import pallas as pl
from jax.experimental.pallas import tpu as pltpu, tpu_sc as plsc

N = 10000
E = 320000
F = 128

NC = 2
NS = 16
L = 16
NW = NC * NS
EPW = E // NW
NG = EPW // L
RING = 5
CBITS = 14
CMASK = (1 << CBITS) - 1


def _dense_body(x_ref, w_ref, as_ref, an_ref, er_ref, ec_ref,
                f_ref, s_ref, n_ref, rc_ref):
    f = jnp.dot(x_ref[...], w_ref[...], preferred_element_type=jnp.float32)
    f_ref[...] = f
    s_ref[...] = jnp.dot(f, as_ref[...], preferred_element_type=jnp.float32)
    n_ref[...] = jnp.dot(f, an_ref[...], preferred_element_type=jnp.float32)
    rc_ref[...] = (er_ref[...] << CBITS) | ec_ref[...]


def _dense(X, W, a_self, a_neigh, er, ec):
    return pl.pallas_call(
        _dense_body,
        out_shape=[
            jax.ShapeDtypeStruct((N, F), jnp.float32),
            jax.ShapeDtypeStruct((N, 1), jnp.float32),
            jax.ShapeDtypeStruct((N, 1), jnp.float32),
            jax.ShapeDtypeStruct((E // F, F), jnp.int32),
        ],
    )(X, W, a_self, a_neigh, er, ec)


def _sc_body(feat_hbm, s_hbm, n_hbm, rc_hbm, zr_hbm, zd_hbm,
             part_hbm, den_hbm,
             s_v, n_v, rc_v, buf, exst, cidx, ridx, out_sp, den_sp, *sems):
    gsems = sems[:RING]
    ssems = sems[RING:2 * RING]
    dsems = sems[2 * RING:]
    cid = lax.axis_index("c")
    tid = lax.axis_index("s")
    wid = tid * NC + cid
    NPT = N // NS

    pltpu.sync_copy(s_hbm, s_v)
    pltpu.sync_copy(n_hbm, n_v)
    pltpu.sync_copy(rc_hbm.at[wid], rc_v)
    pltpu.sync_copy(zr_hbm, out_sp.at[pl.ds(tid * NPT, NPT)])

    @pl.when(tid == 0)
    def _():
        pltpu.sync_copy(zd_hbm, den_sp)

    def _mx(i, m):
        return jnp.maximum(m, n_v[pl.ds(i * L, L)])
    m = lax.fori_loop(0, N // L, _mx, jnp.full((L,), -jnp.inf, jnp.float32))
    lanes = lax.iota(jnp.int32, L)
    for sh in (8, 4, 2, 1):
        m = jnp.maximum(m, m[jnp.bitwise_xor(lanes, sh)])
    max_n = m

    plsc.subcore_barrier()

    for b in range(RING):
        cidx[b, :] = rc_v[b] & CMASK
        pltpu.async_copy(feat_hbm.at[cidx.at[b]], buf.at[b], gsems[b])

    def _group(o, b):
        g = o * RING + b
        pltpu.make_async_copy(
            feat_hbm.at[cidx.at[b]], buf.at[b], gsems[b]).wait()

        pk = rc_v[g]
        rg = pk >> CBITS
        cg = pk & CMASK
        sr = plsc.load_gather(s_v, [rg])
        nc_ = plsc.load_gather(n_v, [cg])
        e = sr + nc_
        e = jnp.where(e > 0, e, 0.2 * e)
        cap = sr + max_n
        cap = jnp.where(cap > 0, cap, 0.2 * cap)
        ex = jnp.exp(e - cap)

        @pl.when(o >= 1)
        def _():
            pltpu.make_async_copy(
                exst.at[b], den_sp.at[ridx.at[b]], dsems[b]).wait()
        exst[b, :] = ex
        ridx[b, :] = rg
        pltpu.async_copy(exst.at[b], den_sp.at[ridx.at[b]], dsems[b],
                         add=True)

        for j in range(L):
            w = ex[jnp.full((L,), j, jnp.int32)]
            for c in range(F // L):
                sl = slice(c * L, (c + 1) * L)
                buf[b, j, sl] = buf[b, j, sl] * w

        pltpu.async_copy(buf.at[b], out_sp.at[ridx.at[b]], ssems[b],
                         add=True)

        bp = (b - 1) % RING
        pg = g + RING - 1
        cond = pg <= NG - 1 if b != 0 else (o >= 1) & (pg <= NG - 1)

        @pl.when(cond)
        def _():
            pltpu.make_async_copy(
                buf.at[bp], out_sp.at[ridx.at[bp]], ssems[bp]).wait()
            cidx[bp, :] = rc_v[pg] & CMASK
            pltpu.async_copy(feat_hbm.at[cidx.at[bp]], buf.at[bp],
                             gsems[bp])

    def _outer(o, carry):
        for b in range(RING):
            _group(o, b)
        return carry

    lax.fori_loop(0, NG // RING, _outer, jnp.int32(0))

    for b in range(RING):
        pltpu.make_async_copy(
            buf.at[b], out_sp.at[ridx.at[b]], ssems[b]).wait()
        pltpu.make_async_copy(
            exst.at[b], den_sp.at[ridx.at[b]], dsems[b]).wait()

    plsc.subcore_barrier()

    pltpu.sync_copy(out_sp.at[pl.ds(tid * NPT, NPT)],
                    part_hbm.at[cid, pl.ds(tid * NPT, NPT)])

    @pl.when(tid == 0)
    def _():
        pltpu.sync_copy(den_sp, den_hbm.at[pl.ds(cid * N, N)])


def _sc_edges(feat, s, n, rc3d, zrows, zden):
    mesh = plsc.VectorSubcoreMesh(core_axis_name="c", subcore_axis_name="s")
    scratch = [
        pltpu.VMEM((N,), jnp.float32),
        pltpu.VMEM((N,), jnp.float32),
        pltpu.VMEM((NG, L), jnp.int32),
        pltpu.VMEM((RING, L, F), jnp.float32),
        pltpu.VMEM((RING, L), jnp.float32),
        pltpu.VMEM((RING, L), jnp.int32),
        pltpu.VMEM((RING, L), jnp.int32),
        pltpu.VMEM_SHARED((N, F), jnp.float32),
        pltpu.VMEM_SHARED((N,), jnp.float32),
    ] + [pltpu.SemaphoreType.DMA] * (3 * RING)
    run = pl.kernel(
        _sc_body,
        out_type=[
            jax.ShapeDtypeStruct((NC, N, F), jnp.float32),
            jax.ShapeDtypeStruct((NC * N,), jnp.float32),
        ],
        mesh=mesh,
        scratch_types=scratch,
        compiler_params=pltpu.CompilerParams(
            needs_layout_passes=False, use_tc_tiling_on_sc=False),
    )
    return run(feat, s, n, rc3d, zrows, zden)


def _combine_body(p_ref, d_ref, b_ref, o_ref):
    ps = p_ref[0] + p_ref[1]
    den = d_ref[0] + d_ref[1]
    o_ref[...] = jnp.maximum(ps / (den[:, None] + 1e-9) + b_ref[...], 0.0)


def _combine(partials, denoms, bias2d):
    return pl.pallas_call(
        _combine_body,
        out_shape=jax.ShapeDtypeStruct((N, F), jnp.float32),
    )(partials, denoms, bias2d)


def kernel(X, edge_index, W, a_self, a_neigh, bias):
    er = edge_index[0].reshape(E // F, F)
    ec = edge_index[1].reshape(E // F, F)
    feat, s2, n2, rc = _dense(X, W, a_self, a_neigh, er, ec)
    s = s2.reshape(N)
    n = n2.reshape(N)
    rc3d = rc.reshape(NW, NG, L)
    zrows = jnp.zeros((N // NS, F), jnp.float32)
    zden = jnp.zeros((N,), jnp.float32)
    partials, denoms = _sc_edges(feat, s, n, rc3d, zrows, zden)
    return _combine(partials, denoms.reshape(NC, N), bias.reshape(1, F))


# speedup vs baseline: 53.0413x; 1.0766x over previous
import jax
import jax.numpy as jnp
from jax import lax
from jax.experimental import pallas as pl
from jax.experimental.pallas import tpu as pltpu, tpu_sc as plsc

N = 10000
E = 320000
F = 128

NC = 2
NS = 16
L = 16
NW = NC * NS
EPW = E // NW
NG = EPW // L
RING = 5
CBITS = 14
CMASK = (1 << CBITS) - 1


def _dense_body(x_ref, w_ref, as_ref, an_ref, e_ref,
                f_ref, s_ref, n_ref, rc_ref):
    f = jnp.dot(x_ref[...], w_ref[...], preferred_element_type=jnp.float32)
    f_ref[...] = f
    s_ref[...] = jnp.dot(f, as_ref[...], preferred_element_type=jnp.float32)
    n_ref[...] = jnp.dot(f, an_ref[...], preferred_element_type=jnp.float32)
    rc_ref[...] = (e_ref[0] << CBITS) | e_ref[1]


def _dense(X, W, a_self, a_neigh, e3d):
    return pl.pallas_call(
        _dense_body,
        out_shape=[
            jax.ShapeDtypeStruct((N, F), jnp.float32),
            jax.ShapeDtypeStruct((N, 1), jnp.float32),
            jax.ShapeDtypeStruct((N, 1), jnp.float32),
            jax.ShapeDtypeStruct((E // F, F), jnp.int32),
        ],
    )(X, W, a_self, a_neigh, e3d)


def _sc_body(feat_hbm, s_hbm, n_hbm, rc_hbm, zd_hbm,
             part_hbm, den_hbm,
             s_v, n_v, rc_v, buf, exst, out_sp, den_sp, *sems):
    gsems = sems[:RING]
    ssems = sems[RING:2 * RING]
    dsems = sems[2 * RING:]
    cid = lax.axis_index("c")
    tid = lax.axis_index("s")
    wid = tid * NC + cid
    NPT = N // NS

    pltpu.sync_copy(s_hbm, s_v)
    pltpu.sync_copy(n_hbm, n_v)
    pltpu.sync_copy(rc_hbm.at[wid], rc_v)
    zv = jnp.zeros((L,), jnp.float32)
    for k in range(RING * L):
        for c in range(F // L):
            buf[k, pl.ds(c * L, L)] = zv
    for k in range(NPT // (RING * L)):
        pltpu.sync_copy(buf, out_sp.at[pl.ds(tid * NPT + k * RING * L,
                                             RING * L)])

    @pl.when(tid == 0)
    def _():
        pltpu.sync_copy(zd_hbm, den_sp)

    def _mx(i, m):
        return jnp.maximum(m, n_v[pl.ds(i * L, L)])
    m = lax.fori_loop(0, N // L, _mx, jnp.full((L,), -jnp.inf, jnp.float32))
    lanes = lax.iota(jnp.int32, L)
    for sh in (8, 4, 2, 1):
        m = jnp.maximum(m, m[jnp.bitwise_xor(lanes, sh)])
    max_n = m

    plsc.subcore_barrier()

    for b in range(RING):
        pltpu.async_copy(feat_hbm.at[rc_v[b] & CMASK],
                         buf.at[pl.ds(b * L, L)], gsems[b])

    def _group(o, b):
        g = o * RING + b
        pk = rc_v[g]
        rg = pk >> CBITS
        cg = pk & CMASK
        pltpu.make_async_copy(
            feat_hbm.at[cg], buf.at[pl.ds(b * L, L)], gsems[b]).wait()
        sr = plsc.load_gather(s_v, [rg])
        nc_ = plsc.load_gather(n_v, [cg])
        e = sr + nc_
        e = jnp.where(e > 0, e, 0.2 * e)
        cap = sr + max_n
        cap = jnp.where(cap > 0, cap, 0.2 * cap)
        ex = jnp.exp(e - cap)

        @pl.when(o >= 1)
        def _():
            pltpu.make_async_copy(
                exst.at[b], den_sp.at[rg], dsems[b]).wait()
        exst[b, :] = ex
        pltpu.async_copy(exst.at[b], den_sp.at[rg], dsems[b], add=True)

        for j in range(L):
            w = ex[jnp.full((L,), j, jnp.int32)]
            for c in range(F // L):
                sl = slice(c * L, (c + 1) * L)
                buf[b * L + j, sl] = buf[b * L + j, sl] * w

        pltpu.async_copy(buf.at[pl.ds(b * L, L)], out_sp.at[rg],
                         ssems[b], add=True)

        bp = (b - 1) % RING
        pg = g + RING - 1
        cond = pg <= NG - 1 if b != 0 else (o >= 1) & (pg <= NG - 1)

        @pl.when(cond)
        def _():
            pltpu.make_async_copy(
                buf.at[pl.ds(bp * L, L)], out_sp.at[rg], ssems[bp]).wait()
            pltpu.async_copy(feat_hbm.at[rc_v[pg] & CMASK],
                             buf.at[pl.ds(bp * L, L)], gsems[bp])

    def _outer(o, carry):
        for b in range(RING):
            _group(o, b)
        return carry

    lax.fori_loop(0, NG // RING, _outer, jnp.int32(0))

    zi = jnp.zeros((L,), jnp.int32)
    for b in range(RING):
        pltpu.make_async_copy(
            buf.at[pl.ds(b * L, L)], out_sp.at[zi], ssems[b]).wait()
        pltpu.make_async_copy(exst.at[b], den_sp.at[zi], dsems[b]).wait()

    plsc.subcore_barrier()

    pltpu.sync_copy(out_sp.at[pl.ds(tid * NPT, NPT)],
                    part_hbm.at[cid, pl.ds(tid * NPT, NPT)])

    @pl.when(tid == 0)
    def _():
        pltpu.sync_copy(den_sp, den_hbm.at[pl.ds(cid * N, N)])


def _sc_edges(feat, s, n, rc3d, zden):
    mesh = plsc.VectorSubcoreMesh(core_axis_name="c", subcore_axis_name="s")
    scratch = [
        pltpu.VMEM((N,), jnp.float32),
        pltpu.VMEM((N,), jnp.float32),
        pltpu.VMEM((NG, L), jnp.int32),
        pltpu.VMEM((RING * L, F), jnp.float32),
        pltpu.VMEM((RING, L), jnp.float32),
        pltpu.VMEM_SHARED((N, F), jnp.float32),
        pltpu.VMEM_SHARED((N,), jnp.float32),
    ] + [pltpu.SemaphoreType.DMA] * (3 * RING)
    run = pl.kernel(
        _sc_body,
        out_type=[
            jax.ShapeDtypeStruct((NC, N, F), jnp.float32),
            jax.ShapeDtypeStruct((NC * N,), jnp.float32),
        ],
        mesh=mesh,
        scratch_types=scratch,
        compiler_params=pltpu.CompilerParams(
            needs_layout_passes=False, use_tc_tiling_on_sc=False),
    )
    return run(feat, s, n, rc3d, zden)


def _combine_body(p_ref, d_ref, b_ref, o_ref):
    ps = p_ref[0] + p_ref[1]
    den = d_ref[0] + d_ref[1]
    o_ref[...] = jnp.maximum(ps / (den[:, None] + 1e-9) + b_ref[...], 0.0)


def _combine(partials, denoms, bias2d):
    return pl.pallas_call(
        _combine_body,
        out_shape=jax.ShapeDtypeStruct((N, F), jnp.float32),
    )(partials, denoms, bias2d)


def kernel(X, edge_index, W, a_self, a_neigh, bias):
    e3d = edge_index.reshape(2, E // F, F)
    feat, s2, n2, rc = _dense(X, W, a_self, a_neigh, e3d)
    s = s2.reshape(N)
    n = n2.reshape(N)
    rc3d = rc.reshape(NW, NG, L)
    zden = jnp.zeros((N,), jnp.float32)
    partials, denoms = _sc_edges(feat, s, n, rc3d, zden)
    return _combine(partials, denoms.reshape(NC, N), bias.reshape(1, F))
